# baseline (speedup 1.0000x reference)
import functools
import jax
import jax.numpy as jnp
from jax import lax
from jax.experimental import pallas as pl
from jax.experimental.pallas import tpu as pltpu
from jax.experimental.pallas import tpu_sc as plsc

M = 1024
D = 512
NA = M + D
S = 8
F = 256
E = 128
H = 256
NR = 4
TOPK = 32
NB = 2048
HS = 64
NEG = -1e30


def _lane_scalar(vec_ref, i):
    lane = lax.broadcasted_iota(jnp.int32, (1, 128), 1)
    return jnp.sum(jnp.where(lane == i, vec_ref[...], 0.0))


def _sparsify_body(sim_ref, o_ref, *, n, br):
    i = pl.program_id(0)
    sim = sim_ref[...]
    col = lax.broadcasted_iota(jnp.int32, (br, n), 1)
    row = lax.broadcasted_iota(jnp.int32, (br, n), 0) + i * br
    diag = col == row
    work0 = jnp.where(diag, NEG, sim)

    def step(_, work):
        mx = jnp.max(work, axis=1, keepdims=True)
        pos = jnp.min(jnp.where(work == mx, col, n), axis=1, keepdims=True)
        return jnp.where(col == pos, NEG, work)

    work = lax.fori_loop(0, TOPK, step, work0)
    sel = (work == NEG) & (~diag)
    o_ref[...] = jnp.where(sel, sim, 0.0)


def _sparsify(sim, n):
    br = 256
    return pl.pallas_call(
        functools.partial(_sparsify_body, n=n, br=br),
        grid=(n // br,),
        in_specs=[pl.BlockSpec((br, n), lambda j: (j, 0))],
        out_specs=pl.BlockSpec((br, n), lambda j: (j, 0)),
        out_shape=jax.ShapeDtypeStruct((n, n), jnp.float32),
    )(sim)


def _featpe_body(spu_ref, sput_ref, wf_ref, we_ref, sp_ref, af_ref, pe_ref):
    sp = jnp.maximum(spu_ref[...], sput_ref[...])
    af = jnp.tanh(jnp.dot(sp, wf_ref[...], preferred_element_type=jnp.float32))
    sp_ref[...] = sp
    af_ref[...] = af
    pe_ref[...] = jnp.dot(af, we_ref[...], preferred_element_type=jnp.float32)


def _featpe(spu, sput, Wf, We_half, n):
    br = 256
    return pl.pallas_call(
        _featpe_body,
        grid=(n // br,),
        in_specs=[
            pl.BlockSpec((br, n), lambda j: (j, 0)),
            pl.BlockSpec((br, n), lambda j: (j, 0)),
            pl.BlockSpec((n, F), lambda j: (0, 0)),
            pl.BlockSpec((F, E), lambda j: (0, 0)),
        ],
        out_specs=[
            pl.BlockSpec((br, n), lambda j: (j, 0)),
            pl.BlockSpec((br, F), lambda j: (j, 0)),
            pl.BlockSpec((br, E), lambda j: (j, 0)),
        ],
        out_shape=[
            jax.ShapeDtypeStruct((n, n), jnp.float32),
            jax.ShapeDtypeStruct((n, F), jnp.float32),
            jax.ShapeDtypeStruct((n, E), jnp.float32),
        ],
    )(spu, sput, Wf, We_half)


def _top8_body(l_ref, r_ref, t8_ref, rel_ref, *, br, roff):
    i = pl.program_id(0)
    x = jnp.concatenate([l_ref[...], r_ref[...]], axis=1)
    col = lax.broadcasted_iota(jnp.int32, (br, NA), 1)
    rowv = lax.broadcasted_iota(jnp.int32, (br, 1), 0) + roff + i * br
    lane = lax.broadcasted_iota(jnp.int32, (br, 128), 1)
    t8 = jnp.zeros((br, 128), jnp.int32)
    rel = jnp.zeros((br, 128), jnp.int32)
    work = x
    for j in range(S):
        mx = jnp.max(work, axis=1, keepdims=True)
        pos = jnp.min(jnp.where(work == mx, col, NA), axis=1, keepdims=True)
        work = jnp.where(col == pos, NEG, work)
        code = jnp.where((rowv < M) & (pos < M), 1,
                         jnp.where((rowv >= M) & (pos >= M), 2, 3))
        code = code * (mx > 0).astype(jnp.int32)
        t8 = t8 + jnp.where(lane == j, pos, 0)
        rel = rel + jnp.where(lane == j, code, 0)
    t8_ref[...] = t8
    rel_ref[...] = rel


def _top8(left, right, nrow, roff):
    br = 128
    return pl.pallas_call(
        functools.partial(_top8_body, br=br, roff=roff),
        grid=(nrow // br,),
        in_specs=[
            pl.BlockSpec((br, M), lambda j: (j, 0)),
            pl.BlockSpec((br, D), lambda j: (j, 0)),
        ],
        out_specs=[
            pl.BlockSpec((br, 128), lambda j: (j, 0)),
            pl.BlockSpec((br, 128), lambda j: (j, 0)),
        ],
        out_shape=[
            jax.ShapeDtypeStruct((nrow, 128), jnp.int32),
            jax.ShapeDtypeStruct((nrow, 128), jnp.int32),
        ],
    )(left, right)


def _leaky(x):
    return jnp.where(x > 0, x, 0.2 * x)


def _agg_body(af_ref, af8_ref, rel_ref, wr_ref, wagg_ref, bagg_ref, o_ref):
    parent = af_ref[...]
    child = af8_ref[...]
    rel = rel_ref[...][:, :S]
    w0 = _lane_scalar(wr_ref, 0)
    w1 = _lane_scalar(wr_ref, 1)
    w2 = _lane_scalar(wr_ref, 2)
    w3 = _lane_scalar(wr_ref, 3)
    wr = (jnp.where(rel == 0, w0, 0.0) + jnp.where(rel == 1, w1, 0.0)
          + jnp.where(rel == 2, w2, 0.0) + jnp.where(rel == 3, w3, 0.0))
    score = jnp.sum(parent[:, None, :] * child, axis=-1) / 16.0 + wr
    score = _leaky(score)
    mx = jnp.max(score, axis=-1, keepdims=True)
    ex = jnp.exp(score - mx)
    att = ex / jnp.sum(ex, axis=-1, keepdims=True)
    upd = jnp.sum(att[..., None] * child, axis=1)
    hid = (jnp.dot(parent, wagg_ref[...][:F, :], preferred_element_type=jnp.float32)
           + jnp.dot(upd, wagg_ref[...][F:, :], preferred_element_type=jnp.float32)
           + bagg_ref[...])
    o_ref[...] = _leaky(hid)


def _agg(all_feat, af8, rel8p, wrelv, W_agg, b_agg):
    br = 256
    return pl.pallas_call(
        _agg_body,
        grid=(NA // br,),
        in_specs=[
            pl.BlockSpec((br, F), lambda j: (j, 0)),
            pl.BlockSpec((br, S, F), lambda j: (j, 0, 0)),
            pl.BlockSpec((br, 128), lambda j: (j, 0)),
            pl.BlockSpec((1, 128), lambda j: (0, 0)),
            pl.BlockSpec((2 * F, F), lambda j: (0, 0)),
            pl.BlockSpec((1, F), lambda j: (0, 0)),
        ],
        out_specs=pl.BlockSpec((br, F), lambda j: (j, 0)),
        out_shape=jax.ShapeDtypeStruct((NA, F), jnp.float32),
    )(all_feat, af8, rel8p, wrelv, W_agg, b_agg)


def _krvr_body(agg8_ref, roh_ref, wk_ref, wv_ref, wkr_ref, wvr_ref, kr_ref, vr_ref):
    a8 = agg8_ref[...]
    roh = roh_ref[...]
    kr = (jnp.dot(a8, wk_ref[...], preferred_element_type=jnp.float32)
          + jnp.dot(roh, wkr_ref[...], preferred_element_type=jnp.float32))
    vr = (jnp.dot(a8, wv_ref[...], preferred_element_type=jnp.float32)
          + jnp.dot(roh, wvr_ref[...], preferred_element_type=jnp.float32))
    kr_ref[...] = kr.astype(jnp.bfloat16)
    vr_ref[...] = vr.astype(jnp.bfloat16)


def _krvr(agg8, roh, Wk4, Wv4, Wkr8, Wvr8):
    nrow = agg8.shape[0]
    br = 512
    return pl.pallas_call(
        _krvr_body,
        grid=(nrow // br,),
        in_specs=[
            pl.BlockSpec((br, F), lambda j: (j, 0)),
            pl.BlockSpec((br, 8), lambda j: (j, 0)),
            pl.BlockSpec((F, H), lambda j: (0, 0)),
            pl.BlockSpec((F, H), lambda j: (0, 0)),
            pl.BlockSpec((8, H), lambda j: (0, 0)),
            pl.BlockSpec((8, H), lambda j: (0, 0)),
        ],
        out_specs=[
            pl.BlockSpec((br, H), lambda j: (j, 0)),
            pl.BlockSpec((br, H), lambda j: (j, 0)),
        ],
        out_shape=[
            jax.ShapeDtypeStruct((nrow, H), jnp.bfloat16),
            jax.ShapeDtypeStruct((nrow, H), jnp.bfloat16),
        ],
    )(agg8, roh, Wk4, Wv4, Wkr8, Wvr8)


def _final_body(km_ref, kd_ref, vm_ref, vd_ref, pem_ref, ped_ref,
                be_ref, wq_ref, va_ref, ws1_ref, ws2_ref, bs_ref, o_ref):
    edge = jnp.maximum(pem_ref[...] + ped_ref[...] + be_ref[...], 0.0)
    q = jnp.dot(edge, wq_ref[...], preferred_element_type=jnp.float32)
    va = va_ref[...]
    es = []
    for ref in (km_ref, kd_ref):
        for j in range(S):
            kj = ref[:, j * H:(j + 1) * H].astype(jnp.float32)
            es.append(jnp.sum(jnp.tanh(q + kj) * va, axis=-1, keepdims=True))
    e = jnp.concatenate(es, axis=1)
    mx = jnp.max(e, axis=-1, keepdims=True)
    ex = jnp.exp(e - mx)
    a = ex / jnp.sum(ex, axis=-1, keepdims=True)
    info = jnp.zeros_like(q)
    for r, ref in enumerate((vm_ref, vd_ref)):
        for j in range(S):
            vj = ref[:, j * H:(j + 1) * H].astype(jnp.float32)
            info = info + a[:, r * S + j:r * S + j + 1] * vj
    val = (jnp.sum(edge * ws1_ref[...], axis=-1)
           + jnp.sum(info * ws2_ref[...], axis=-1)
           + _lane_scalar(bs_ref, 0))
    o_ref[...] = jnp.broadcast_to(val[:, None], o_ref.shape)


def _final(kk, vv, pepe, b_edge, Wq, v_att, ws1, ws2, bsv):
    bb = 128
    nblk = NB // bb
    spec2m = pl.BlockSpec((bb, S * H), lambda j: (j, 0))
    spec2d = pl.BlockSpec((bb, S * H), lambda j: (j + nblk, 0))
    spec_pm = pl.BlockSpec((bb, E), lambda j: (j, 0))
    spec_pd = pl.BlockSpec((bb, E), lambda j: (j + nblk, 0))

    def full(shp):
        return pl.BlockSpec(shp, lambda j, _n=len(shp): (0,) * _n)

    return pl.pallas_call(
        _final_body,
        grid=(nblk,),
        in_specs=[spec2m, spec2d, spec2m, spec2d, spec_pm, spec_pd,
                  full((1, E)), full((E, H)), full((1, H)),
                  full((1, E)), full((1, H)), full((1, 128))],
        out_specs=pl.BlockSpec((bb, 128), lambda j: (j, 0)),
        out_shape=jax.ShapeDtypeStruct((NB, 128), jnp.float32),
    )(kk, kk, vv, vv, pepe, pepe, b_edge.reshape(1, E), Wq,
      v_att.reshape(1, H), ws1, ws2, bsv)


def _gather_rows(table, idx):
    nrow, d = table.shape
    m = idx.shape[0]
    nw = 32
    bpw = m // nw
    ch = bpw
    while ch * d * 4 > 131072:
        ch //= 2
    nch = bpw // ch
    mesh = plsc.VectorSubcoreMesh(core_axis_name="c", subcore_axis_name="s")

    @functools.partial(
        pl.kernel, mesh=mesh,
        out_type=jax.ShapeDtypeStruct((m, d), table.dtype),
        scratch_types=[
            pltpu.VMEM((ch,), jnp.int32),
            pltpu.VMEM((ch, d), table.dtype),
            pltpu.VMEM((ch, d), table.dtype),
            pltpu.SemaphoreType.DMA,
            pltpu.SemaphoreType.DMA,
            pltpu.SemaphoreType.DMA,
        ],
    )
    def k(tab_hbm, idx_hbm, out_hbm, idx_v, rows0, rows1, gsem, ws0, ws1):
        wid = lax.axis_index("s") * 2 + lax.axis_index("c")
        base = wid * bpw
        bufs = (rows0, rows1)
        wsems = (ws0, ws1)
        wb = [None, None]
        for c in range(nch):
            b = c % 2
            off = base + c * ch
            if wb[b] is not None:
                wb[b].wait()
            pltpu.sync_copy(idx_hbm.at[pl.ds(off, ch)], idx_v)
            pltpu.async_copy(tab_hbm.at[idx_v], bufs[b], gsem).wait()
            wb[b] = pltpu.async_copy(bufs[b], out_hbm.at[pl.ds(off, ch)],
                                     wsems[b])
        for b in range(2):
            if wb[b] is not None:
                wb[b].wait()

    return k(table, idx)


def kernel(mm_f, mm_s, mm_g, dd_t, dd_s, dd_g, m_d, md_node, Wp_m, c_m, Wp_d,
           c_d, Wm, Wd, w_rel, W_agg, b_agg, W_edge, b_edge, Wq, Wk, Wv, v_att,
           W_s, b_s):
    def pad128(v):
        return jnp.zeros((1, 128), jnp.float32).at[0, :v.shape[0]].set(v)

    def _view_combine(views, Wp, c):
        stack = jnp.stack(views)
        h = jnp.tanh(jnp.einsum('vij,jk->vik', stack, Wp))
        s = jnp.mean(jnp.einsum('vik,k->vi', h, c), axis=1)
        return jnp.einsum('v,vij->ij', jax.nn.softmax(s), stack)

    spu_m = _sparsify(_view_combine((mm_f, mm_s, mm_g), Wp_m, c_m), M)
    spu_d = _sparsify(_view_combine((dd_t, dd_s, dd_g), Wp_d, c_d), D)

    sp_m, af_m, pe_m = _featpe(spu_m, spu_m.T, Wm, W_edge[:F], M)
    sp_d, af_d, pe_d = _featpe(spu_d, spu_d.T, Wd, W_edge[F:], D)
    all_feat = jnp.concatenate([af_m, af_d], 0)
    pe = jnp.concatenate([pe_m, pe_d], 0)

    t8m, rel8m = _top8(sp_m, m_d, M, 0)
    t8d, rel8d = _top8(m_d.T, sp_d, D, M)
    t8p = jnp.concatenate([t8m, t8d], 0)
    rel8p = jnp.concatenate([rel8m, rel8d], 0)
    t8f = t8p[:, :S].reshape(NA * S)

    af8 = _gather_rows(all_feat, t8f).reshape(NA, S, F)
    agg = _agg(all_feat, af8, rel8p, pad128(w_rel), W_agg, b_agg.reshape(1, F))

    NU = D + D
    t8u = jnp.concatenate([t8p[:D, :S], t8p[M:, :S]], 0).reshape(NU * S)
    rel8u = jnp.concatenate([rel8p[:D, :S], rel8p[M:, :S]], 0).reshape(NU * S)
    agg8 = _gather_rows(agg, t8u)
    roh = jax.nn.one_hot(rel8u, 8, dtype=jnp.float32)
    wkr8 = jnp.zeros((8, H), jnp.float32).at[:NR].set(Wk[:NR])
    wvr8 = jnp.zeros((8, H), jnp.float32).at[:NR].set(Wv[:NR])
    kr, vr = _krvr(agg8, roh, Wk[NR:], Wv[NR:], wkr8, wvr8)

    def pack32(x, nr):
        return lax.bitcast_convert_type(
            x.reshape(nr, S * H // 2, 2), jnp.int32)

    def unpack32(x):
        return lax.bitcast_convert_type(
            x, jnp.bfloat16).reshape(x.shape[0], S * H)

    kr2 = pack32(kr, NU)
    vr2 = pack32(vr, NU)
    peu = jnp.concatenate([pe[:D], pe[M:]], 0)

    idx = jnp.concatenate([md_node[:, 0], md_node[:, 1] + D])
    kk = unpack32(_gather_rows(kr2, idx))
    vv = unpack32(_gather_rows(vr2, idx))
    pepe = _gather_rows(peu, idx)

    out = _final(kk, vv, pepe, b_edge, Wq, v_att,
                 W_s[:E].reshape(1, E), W_s[E:].reshape(1, H), pad128(b_s))
    return out[:, 0]


# speedup vs baseline: 5.1125x; 5.1125x over previous
import functools
import jax
import jax.numpy as jnp
from jax import lax
from jax.experimental import pallas as pl
from jax.experimental.pallas import tpu as pltpu
from jax.experimental.pallas import tpu_sc as plsc

M = 1024
D = 512
NA = M + D
S = 8
F = 256
E = 128
H = 256
NR = 4
TOPK = 32
NB = 2048
HS = 64
NEG = -1e30


def _lane_scalar(vec_ref, i):
    lane = lax.broadcasted_iota(jnp.int32, (1, 128), 1)
    return jnp.sum(jnp.where(lane == i, vec_ref[...], 0.0))


def _sparsify_body(sim_ref, o_ref, *, n, br):
    i = pl.program_id(0)
    sim = sim_ref[...]
    col = lax.broadcasted_iota(jnp.int32, (br, n), 1)
    row = lax.broadcasted_iota(jnp.int32, (br, n), 0) + i * br
    diag = col == row
    work0 = jnp.where(diag, NEG, sim)

    def step(_, work):
        mx = jnp.max(work, axis=1, keepdims=True)
        pos = jnp.min(jnp.where(work == mx, col, n), axis=1, keepdims=True)
        return jnp.where(col == pos, NEG, work)

    work = lax.fori_loop(0, TOPK, step, work0)
    sel = (work == NEG) & (~diag)
    o_ref[...] = jnp.where(sel, sim, 0.0)


def _sparsify(sim, n):
    br = 256
    return pl.pallas_call(
        functools.partial(_sparsify_body, n=n, br=br),
        grid=(n // br,),
        in_specs=[pl.BlockSpec((br, n), lambda j: (j, 0))],
        out_specs=pl.BlockSpec((br, n), lambda j: (j, 0)),
        out_shape=jax.ShapeDtypeStruct((n, n), jnp.float32),
    )(sim)


def _featpe_body(spu_ref, sput_ref, wf_ref, we_ref, sp_ref, af_ref, pe_ref):
    sp = jnp.maximum(spu_ref[...], sput_ref[...])
    af = jnp.tanh(jnp.dot(sp, wf_ref[...], preferred_element_type=jnp.float32))
    sp_ref[...] = sp
    af_ref[...] = af
    pe_ref[...] = jnp.dot(af, we_ref[...], preferred_element_type=jnp.float32)


def _featpe(spu, sput, Wf, We_half, n):
    br = 256
    return pl.pallas_call(
        _featpe_body,
        grid=(n // br,),
        in_specs=[
            pl.BlockSpec((br, n), lambda j: (j, 0)),
            pl.BlockSpec((br, n), lambda j: (j, 0)),
            pl.BlockSpec((n, F), lambda j: (0, 0)),
            pl.BlockSpec((F, E), lambda j: (0, 0)),
        ],
        out_specs=[
            pl.BlockSpec((br, n), lambda j: (j, 0)),
            pl.BlockSpec((br, F), lambda j: (j, 0)),
            pl.BlockSpec((br, E), lambda j: (j, 0)),
        ],
        out_shape=[
            jax.ShapeDtypeStruct((n, n), jnp.float32),
            jax.ShapeDtypeStruct((n, F), jnp.float32),
            jax.ShapeDtypeStruct((n, E), jnp.float32),
        ],
    )(spu, sput, Wf, We_half)


def _top8_body(l_ref, r_ref, t8_ref, rel_ref, *, br, roff):
    i = pl.program_id(0)
    x = jnp.concatenate([l_ref[...], r_ref[...]], axis=1)
    col = lax.broadcasted_iota(jnp.int32, (br, NA), 1)
    rowv = lax.broadcasted_iota(jnp.int32, (br, 1), 0) + roff + i * br
    lane = lax.broadcasted_iota(jnp.int32, (br, 128), 1)
    t8 = jnp.zeros((br, 128), jnp.int32)
    rel = jnp.zeros((br, 128), jnp.int32)
    work = x
    for j in range(S):
        mx = jnp.max(work, axis=1, keepdims=True)
        pos = jnp.min(jnp.where(work == mx, col, NA), axis=1, keepdims=True)
        work = jnp.where(col == pos, NEG, work)
        code = jnp.where((rowv < M) & (pos < M), 1,
                         jnp.where((rowv >= M) & (pos >= M), 2, 3))
        code = code * (mx > 0).astype(jnp.int32)
        t8 = t8 + jnp.where(lane == j, pos, 0)
        rel = rel + jnp.where(lane == j, code, 0)
    t8_ref[...] = t8
    rel_ref[...] = rel


def _top8(left, right, nrow, roff):
    br = 128
    return pl.pallas_call(
        functools.partial(_top8_body, br=br, roff=roff),
        grid=(nrow // br,),
        in_specs=[
            pl.BlockSpec((br, M), lambda j: (j, 0)),
            pl.BlockSpec((br, D), lambda j: (j, 0)),
        ],
        out_specs=[
            pl.BlockSpec((br, 128), lambda j: (j, 0)),
            pl.BlockSpec((br, 128), lambda j: (j, 0)),
        ],
        out_shape=[
            jax.ShapeDtypeStruct((nrow, 128), jnp.int32),
            jax.ShapeDtypeStruct((nrow, 128), jnp.int32),
        ],
    )(left, right)


def _leaky(x):
    return jnp.where(x > 0, x, 0.2 * x)


def _agg_body(af_ref, af8_ref, rel_ref, wr_ref, wagg_ref, bagg_ref, o_ref):
    parent = af_ref[...]
    child = af8_ref[...]
    rel = rel_ref[...][:, :S]
    w0 = _lane_scalar(wr_ref, 0)
    w1 = _lane_scalar(wr_ref, 1)
    w2 = _lane_scalar(wr_ref, 2)
    w3 = _lane_scalar(wr_ref, 3)
    wr = (jnp.where(rel == 0, w0, 0.0) + jnp.where(rel == 1, w1, 0.0)
          + jnp.where(rel == 2, w2, 0.0) + jnp.where(rel == 3, w3, 0.0))
    score = jnp.sum(parent[:, None, :] * child, axis=-1) / 16.0 + wr
    score = _leaky(score)
    mx = jnp.max(score, axis=-1, keepdims=True)
    ex = jnp.exp(score - mx)
    att = ex / jnp.sum(ex, axis=-1, keepdims=True)
    upd = jnp.sum(att[..., None] * child, axis=1)
    hid = (jnp.dot(parent, wagg_ref[...][:F, :], preferred_element_type=jnp.float32)
           + jnp.dot(upd, wagg_ref[...][F:, :], preferred_element_type=jnp.float32)
           + bagg_ref[...])
    o_ref[...] = _leaky(hid)


def _agg(all_feat, af8, rel8p, wrelv, W_agg, b_agg):
    br = 256
    return pl.pallas_call(
        _agg_body,
        grid=(NA // br,),
        in_specs=[
            pl.BlockSpec((br, F), lambda j: (j, 0)),
            pl.BlockSpec((br, S, F), lambda j: (j, 0, 0)),
            pl.BlockSpec((br, 128), lambda j: (j, 0)),
            pl.BlockSpec((1, 128), lambda j: (0, 0)),
            pl.BlockSpec((2 * F, F), lambda j: (0, 0)),
            pl.BlockSpec((1, F), lambda j: (0, 0)),
        ],
        out_specs=pl.BlockSpec((br, F), lambda j: (j, 0)),
        out_shape=jax.ShapeDtypeStruct((NA, F), jnp.float32),
    )(all_feat, af8, rel8p, wrelv, W_agg, b_agg)


def _krvr_body(agg8_ref, roh_ref, wk_ref, wv_ref, wkr_ref, wvr_ref, kr_ref, vr_ref):
    a8 = agg8_ref[...]
    roh = roh_ref[...]
    kr_ref[...] = (jnp.dot(a8, wk_ref[...], preferred_element_type=jnp.float32)
                   + jnp.dot(roh, wkr_ref[...], preferred_element_type=jnp.float32))
    vr_ref[...] = (jnp.dot(a8, wv_ref[...], preferred_element_type=jnp.float32)
                   + jnp.dot(roh, wvr_ref[...], preferred_element_type=jnp.float32))


def _krvr(agg8, roh, Wk4, Wv4, Wkr8, Wvr8):
    nrow = agg8.shape[0]
    br = 512
    return pl.pallas_call(
        _krvr_body,
        grid=(nrow // br,),
        in_specs=[
            pl.BlockSpec((br, F), lambda j: (j, 0)),
            pl.BlockSpec((br, 8), lambda j: (j, 0)),
            pl.BlockSpec((F, H), lambda j: (0, 0)),
            pl.BlockSpec((F, H), lambda j: (0, 0)),
            pl.BlockSpec((8, H), lambda j: (0, 0)),
            pl.BlockSpec((8, H), lambda j: (0, 0)),
        ],
        out_specs=[
            pl.BlockSpec((br, H), lambda j: (j, 0)),
            pl.BlockSpec((br, H), lambda j: (j, 0)),
        ],
        out_shape=[
            jax.ShapeDtypeStruct((nrow, H), jnp.float32),
            jax.ShapeDtypeStruct((nrow, H), jnp.float32),
        ],
    )(agg8, roh, Wk4, Wv4, Wkr8, Wvr8)


def _final_body(km_ref, kd_ref, vm_ref, vd_ref, pem_ref, ped_ref,
                be_ref, wq_ref, va_ref, ws1_ref, ws2_ref, bs_ref, o_ref):
    edge = jnp.maximum(pem_ref[...] + ped_ref[...] + be_ref[...], 0.0)
    q = jnp.dot(edge, wq_ref[...], preferred_element_type=jnp.float32)
    va = va_ref[...]
    es = []
    for ref in (km_ref, kd_ref):
        for j in range(S):
            kj = ref[:, j * H:(j + 1) * H]
            es.append(jnp.sum(jnp.tanh(q + kj) * va, axis=-1, keepdims=True))
    e = jnp.concatenate(es, axis=1)
    mx = jnp.max(e, axis=-1, keepdims=True)
    ex = jnp.exp(e - mx)
    a = ex / jnp.sum(ex, axis=-1, keepdims=True)
    info = jnp.zeros_like(q)
    for r, ref in enumerate((vm_ref, vd_ref)):
        for j in range(S):
            info = info + a[:, r * S + j:r * S + j + 1] * ref[:, j * H:(j + 1) * H]
    val = (jnp.sum(edge * ws1_ref[...], axis=-1)
           + jnp.sum(info * ws2_ref[...], axis=-1)
           + _lane_scalar(bs_ref, 0))
    o_ref[...] = jnp.broadcast_to(val[:, None], o_ref.shape)


def _final(kk, vv, pepe, b_edge, Wq, v_att, ws1, ws2, bsv):
    bb = 128
    nblk = NB // bb
    spec2m = pl.BlockSpec((bb, S * H), lambda j: (j, 0))
    spec2d = pl.BlockSpec((bb, S * H), lambda j: (j + nblk, 0))
    spec_pm = pl.BlockSpec((bb, E), lambda j: (j, 0))
    spec_pd = pl.BlockSpec((bb, E), lambda j: (j + nblk, 0))

    def full(shp):
        return pl.BlockSpec(shp, lambda j, _n=len(shp): (0,) * _n)

    return pl.pallas_call(
        _final_body,
        grid=(nblk,),
        in_specs=[spec2m, spec2d, spec2m, spec2d, spec_pm, spec_pd,
                  full((1, E)), full((E, H)), full((1, H)),
                  full((1, E)), full((1, H)), full((1, 128))],
        out_specs=pl.BlockSpec((bb, 128), lambda j: (j, 0)),
        out_shape=jax.ShapeDtypeStruct((NB, 128), jnp.float32),
    )(kk, kk, vv, vv, pepe, pepe, b_edge.reshape(1, E), Wq,
      v_att.reshape(1, H), ws1, ws2, bsv)


def _gather_rows(table, idx):
    nrow, d = table.shape
    m = idx.shape[0]
    nw = 32
    bpw = m // nw
    ch = bpw
    while ch * d * 4 > 131072:
        ch //= 2
    nch = bpw // ch
    mesh = plsc.VectorSubcoreMesh(core_axis_name="c", subcore_axis_name="s")

    @functools.partial(
        pl.kernel, mesh=mesh,
        out_type=jax.ShapeDtypeStruct((m, d), table.dtype),
        scratch_types=[
            pltpu.VMEM((ch,), jnp.int32),
            pltpu.VMEM((ch, d), table.dtype),
            pltpu.VMEM((ch, d), table.dtype),
            pltpu.SemaphoreType.DMA,
            pltpu.SemaphoreType.DMA,
            pltpu.SemaphoreType.DMA,
        ],
    )
    def k(tab_hbm, idx_hbm, out_hbm, idx_v, rows0, rows1, gsem, ws0, ws1):
        wid = lax.axis_index("s") * 2 + lax.axis_index("c")
        base = wid * bpw
        bufs = (rows0, rows1)
        wsems = (ws0, ws1)
        wb = [None, None]
        for c in range(nch):
            b = c % 2
            off = base + c * ch
            if wb[b] is not None:
                wb[b].wait()
            pltpu.sync_copy(idx_hbm.at[pl.ds(off, ch)], idx_v)
            pltpu.async_copy(tab_hbm.at[idx_v], bufs[b], gsem).wait()
            wb[b] = pltpu.async_copy(bufs[b], out_hbm.at[pl.ds(off, ch)],
                                     wsems[b])
        for b in range(2):
            if wb[b] is not None:
                wb[b].wait()

    return k(table, idx)


def kernel(mm_f, mm_s, mm_g, dd_t, dd_s, dd_g, m_d, md_node, Wp_m, c_m, Wp_d,
           c_d, Wm, Wd, w_rel, W_agg, b_agg, W_edge, b_edge, Wq, Wk, Wv, v_att,
           W_s, b_s):
    def pad128(v):
        return jnp.zeros((1, 128), jnp.float32).at[0, :v.shape[0]].set(v)

    def _view_combine(views, Wp, c):
        stack = jnp.stack(views)
        h = jnp.tanh(jnp.einsum('vij,jk->vik', stack, Wp))
        s = jnp.mean(jnp.einsum('vik,k->vi', h, c), axis=1)
        return jnp.einsum('v,vij->ij', jax.nn.softmax(s), stack)

    spu_m = _sparsify(_view_combine((mm_f, mm_s, mm_g), Wp_m, c_m), M)
    spu_d = _sparsify(_view_combine((dd_t, dd_s, dd_g), Wp_d, c_d), D)

    sp_m, af_m, pe_m = _featpe(spu_m, spu_m.T, Wm, W_edge[:F], M)
    sp_d, af_d, pe_d = _featpe(spu_d, spu_d.T, Wd, W_edge[F:], D)
    all_feat = jnp.concatenate([af_m, af_d], 0)
    pe = jnp.concatenate([pe_m, pe_d], 0)

    t8m, rel8m = _top8(sp_m, m_d, M, 0)
    t8d, rel8d = _top8(m_d.T, sp_d, D, M)
    t8p = jnp.concatenate([t8m, t8d], 0)
    rel8p = jnp.concatenate([rel8m, rel8d], 0)
    t8f = t8p[:, :S].reshape(NA * S)

    af8 = _gather_rows(all_feat, t8f).reshape(NA, S, F)
    agg = _agg(all_feat, af8, rel8p, pad128(w_rel), W_agg, b_agg.reshape(1, F))

    NU = D + D
    t8u = jnp.concatenate([t8p[:D, :S], t8p[M:, :S]], 0).reshape(NU * S)
    rel8u = jnp.concatenate([rel8p[:D, :S], rel8p[M:, :S]], 0).reshape(NU * S)
    agg8 = _gather_rows(agg, t8u)
    roh = jax.nn.one_hot(rel8u, 8, dtype=jnp.float32)
    wkr8 = jnp.zeros((8, H), jnp.float32).at[:NR].set(Wk[:NR])
    wvr8 = jnp.zeros((8, H), jnp.float32).at[:NR].set(Wv[:NR])
    kr, vr = _krvr(agg8, roh, Wk[NR:], Wv[NR:], wkr8, wvr8)
    kr2 = kr.reshape(NU, S * H)
    vr2 = vr.reshape(NU, S * H)
    peu = jnp.concatenate([pe[:D], pe[M:]], 0)

    idx = jnp.concatenate([md_node[:, 0], md_node[:, 1] + D])
    kk = _gather_rows(kr2, idx)
    vv = _gather_rows(vr2, idx)
    pepe = _gather_rows(peu, idx)

    out = _final(kk, vv, pepe, b_edge, Wq, v_att,
                 W_s[:E].reshape(1, E), W_s[E:].reshape(1, H), pad128(b_s))
    return out[:, 0]


# speedup vs baseline: 5.1232x; 1.0021x over previous
import functools
import jax
import jax.numpy as jnp
from jax import lax
from jax.experimental import pallas as pl
from jax.experimental.pallas import tpu as pltpu
from jax.experimental.pallas import tpu_sc as plsc

M = 1024
D = 512
NA = M + D
S = 8
F = 256
E = 128
H = 256
NR = 4
TOPK = 32
NB = 2048
HS = 64
NEG = -1e30


def _lane_scalar(vec_ref, i):
    lane = lax.broadcasted_iota(jnp.int32, (1, 128), 1)
    return jnp.sum(jnp.where(lane == i, vec_ref[...], 0.0))


def _sparsify_body(sim_ref, o_ref, *, n, br):
    i = pl.program_id(0)
    sim = sim_ref[...]
    col = lax.broadcasted_iota(jnp.int32, (br, n), 1)
    row = lax.broadcasted_iota(jnp.int32, (br, n), 0) + i * br
    diag = col == row
    work0 = jnp.where(diag, NEG, sim)

    def step(_, work):
        mx = jnp.max(work, axis=1, keepdims=True)
        pos = jnp.min(jnp.where(work == mx, col, n), axis=1, keepdims=True)
        return jnp.where(col == pos, NEG, work)

    work = lax.fori_loop(0, TOPK, step, work0)
    sel = (work == NEG) & (~diag)
    o_ref[...] = jnp.where(sel, sim, 0.0)


def _sparsify(sim, n):
    br = 256
    return pl.pallas_call(
        functools.partial(_sparsify_body, n=n, br=br),
        grid=(n // br,),
        in_specs=[pl.BlockSpec((br, n), lambda j: (j, 0))],
        out_specs=pl.BlockSpec((br, n), lambda j: (j, 0)),
        out_shape=jax.ShapeDtypeStruct((n, n), jnp.float32),
    )(sim)


def _featpe_body(spu_ref, sput_ref, wf_ref, we_ref, sp_ref, af_ref, pe_ref):
    sp = jnp.maximum(spu_ref[...], sput_ref[...])
    af = jnp.tanh(jnp.dot(sp, wf_ref[...], preferred_element_type=jnp.float32))
    sp_ref[...] = sp
    af_ref[...] = af
    pe_ref[...] = jnp.dot(af, we_ref[...], preferred_element_type=jnp.float32)


def _featpe(spu, sput, Wf, We_half, n):
    br = 256
    return pl.pallas_call(
        _featpe_body,
        grid=(n // br,),
        in_specs=[
            pl.BlockSpec((br, n), lambda j: (j, 0)),
            pl.BlockSpec((br, n), lambda j: (j, 0)),
            pl.BlockSpec((n, F), lambda j: (0, 0)),
            pl.BlockSpec((F, E), lambda j: (0, 0)),
        ],
        out_specs=[
            pl.BlockSpec((br, n), lambda j: (j, 0)),
            pl.BlockSpec((br, F), lambda j: (j, 0)),
            pl.BlockSpec((br, E), lambda j: (j, 0)),
        ],
        out_shape=[
            jax.ShapeDtypeStruct((n, n), jnp.float32),
            jax.ShapeDtypeStruct((n, F), jnp.float32),
            jax.ShapeDtypeStruct((n, E), jnp.float32),
        ],
    )(spu, sput, Wf, We_half)


def _top8_body(l_ref, r_ref, t8_ref, rel_ref, *, br, roff):
    i = pl.program_id(0)
    x = jnp.concatenate([l_ref[...], r_ref[...]], axis=1)
    col = lax.broadcasted_iota(jnp.int32, (br, NA), 1)
    rowv = lax.broadcasted_iota(jnp.int32, (br, 1), 0) + roff + i * br
    lane = lax.broadcasted_iota(jnp.int32, (br, 128), 1)
    t8 = jnp.zeros((br, 128), jnp.int32)
    rel = jnp.zeros((br, 128), jnp.int32)
    work = x
    for j in range(S):
        mx = jnp.max(work, axis=1, keepdims=True)
        pos = jnp.min(jnp.where(work == mx, col, NA), axis=1, keepdims=True)
        work = jnp.where(col == pos, NEG, work)
        code = jnp.where((rowv < M) & (pos < M), 1,
                         jnp.where((rowv >= M) & (pos >= M), 2, 3))
        code = code * (mx > 0).astype(jnp.int32)
        t8 = t8 + jnp.where(lane == j, pos, 0)
        rel = rel + jnp.where(lane == j, code, 0)
    t8_ref[...] = t8
    rel_ref[...] = rel


def _top8(left, right, nrow, roff):
    br = 128
    return pl.pallas_call(
        functools.partial(_top8_body, br=br, roff=roff),
        grid=(nrow // br,),
        in_specs=[
            pl.BlockSpec((br, M), lambda j: (j, 0)),
            pl.BlockSpec((br, D), lambda j: (j, 0)),
        ],
        out_specs=[
            pl.BlockSpec((br, 128), lambda j: (j, 0)),
            pl.BlockSpec((br, 128), lambda j: (j, 0)),
        ],
        out_shape=[
            jax.ShapeDtypeStruct((nrow, 128), jnp.int32),
            jax.ShapeDtypeStruct((nrow, 128), jnp.int32),
        ],
    )(left, right)


def _leaky(x):
    return jnp.where(x > 0, x, 0.2 * x)


def _agg_body(af_ref, af8_ref, rel_ref, wr_ref, wagg_ref, bagg_ref, o_ref):
    parent = af_ref[...]
    child = af8_ref[...]
    rel = rel_ref[...][:, :S]
    w0 = _lane_scalar(wr_ref, 0)
    w1 = _lane_scalar(wr_ref, 1)
    w2 = _lane_scalar(wr_ref, 2)
    w3 = _lane_scalar(wr_ref, 3)
    wr = (jnp.where(rel == 0, w0, 0.0) + jnp.where(rel == 1, w1, 0.0)
          + jnp.where(rel == 2, w2, 0.0) + jnp.where(rel == 3, w3, 0.0))
    score = jnp.sum(parent[:, None, :] * child, axis=-1) / 16.0 + wr
    score = _leaky(score)
    mx = jnp.max(score, axis=-1, keepdims=True)
    ex = jnp.exp(score - mx)
    att = ex / jnp.sum(ex, axis=-1, keepdims=True)
    upd = jnp.sum(att[..., None] * child, axis=1)
    hid = (jnp.dot(parent, wagg_ref[...][:F, :], preferred_element_type=jnp.float32)
           + jnp.dot(upd, wagg_ref[...][F:, :], preferred_element_type=jnp.float32)
           + bagg_ref[...])
    o_ref[...] = _leaky(hid)


def _agg(all_feat, af8, rel8p, wrelv, W_agg, b_agg):
    br = 256
    return pl.pallas_call(
        _agg_body,
        grid=(NA // br,),
        in_specs=[
            pl.BlockSpec((br, F), lambda j: (j, 0)),
            pl.BlockSpec((br, S, F), lambda j: (j, 0, 0)),
            pl.BlockSpec((br, 128), lambda j: (j, 0)),
            pl.BlockSpec((1, 128), lambda j: (0, 0)),
            pl.BlockSpec((2 * F, F), lambda j: (0, 0)),
            pl.BlockSpec((1, F), lambda j: (0, 0)),
        ],
        out_specs=pl.BlockSpec((br, F), lambda j: (j, 0)),
        out_shape=jax.ShapeDtypeStruct((NA, F), jnp.float32),
    )(all_feat, af8, rel8p, wrelv, W_agg, b_agg)


def _krvr_body(agg8_ref, roh_ref, wk_ref, wv_ref, wkr_ref, wvr_ref, kr_ref, vr_ref):
    a8 = agg8_ref[...]
    roh = roh_ref[...]
    kr_ref[...] = (jnp.dot(a8, wk_ref[...], preferred_element_type=jnp.float32)
                   + jnp.dot(roh, wkr_ref[...], preferred_element_type=jnp.float32))
    vr_ref[...] = (jnp.dot(a8, wv_ref[...], preferred_element_type=jnp.float32)
                   + jnp.dot(roh, wvr_ref[...], preferred_element_type=jnp.float32))


def _krvr(agg8, roh, Wk4, Wv4, Wkr8, Wvr8):
    nrow = agg8.shape[0]
    br = 512
    return pl.pallas_call(
        _krvr_body,
        grid=(nrow // br,),
        in_specs=[
            pl.BlockSpec((br, F), lambda j: (j, 0)),
            pl.BlockSpec((br, 8), lambda j: (j, 0)),
            pl.BlockSpec((F, H), lambda j: (0, 0)),
            pl.BlockSpec((F, H), lambda j: (0, 0)),
            pl.BlockSpec((8, H), lambda j: (0, 0)),
            pl.BlockSpec((8, H), lambda j: (0, 0)),
        ],
        out_specs=[
            pl.BlockSpec((br, H), lambda j: (j, 0)),
            pl.BlockSpec((br, H), lambda j: (j, 0)),
        ],
        out_shape=[
            jax.ShapeDtypeStruct((nrow, H), jnp.float32),
            jax.ShapeDtypeStruct((nrow, H), jnp.float32),
        ],
    )(agg8, roh, Wk4, Wv4, Wkr8, Wvr8)


def _final_body(km_ref, kd_ref, vm_ref, vd_ref, pem_ref, ped_ref,
                be_ref, wq_ref, va_ref, ws1_ref, ws2_ref, bs_ref, o_ref):
    edge = jnp.maximum(pem_ref[...] + ped_ref[...] + be_ref[...], 0.0)
    q = jnp.dot(edge, wq_ref[...], preferred_element_type=jnp.float32)
    va = va_ref[...]
    es = []
    for ref in (km_ref, kd_ref):
        for j in range(S):
            kj = ref[:, j * H:(j + 1) * H]
            es.append(jnp.sum(jnp.tanh(q + kj) * va, axis=-1, keepdims=True))
    e = jnp.concatenate(es, axis=1)
    mx = jnp.max(e, axis=-1, keepdims=True)
    ex = jnp.exp(e - mx)
    a = ex / jnp.sum(ex, axis=-1, keepdims=True)
    info = jnp.zeros_like(q)
    for r, ref in enumerate((vm_ref, vd_ref)):
        for j in range(S):
            info = info + a[:, r * S + j:r * S + j + 1] * ref[:, j * H:(j + 1) * H]
    val = (jnp.sum(edge * ws1_ref[...], axis=-1)
           + jnp.sum(info * ws2_ref[...], axis=-1)
           + _lane_scalar(bs_ref, 0))
    o_ref[...] = jnp.broadcast_to(val[:, None], o_ref.shape)


def _final(kk, vv, pepe, b_edge, Wq, v_att, ws1, ws2, bsv):
    bb = 128
    nblk = NB // bb
    spec2m = pl.BlockSpec((bb, S * H), lambda j: (j, 0))
    spec2d = pl.BlockSpec((bb, S * H), lambda j: (j + nblk, 0))
    spec_pm = pl.BlockSpec((bb, E), lambda j: (j, 0))
    spec_pd = pl.BlockSpec((bb, E), lambda j: (j + nblk, 0))

    def full(shp):
        return pl.BlockSpec(shp, lambda j, _n=len(shp): (0,) * _n)

    return pl.pallas_call(
        _final_body,
        grid=(nblk,),
        in_specs=[spec2m, spec2d, spec2m, spec2d, spec_pm, spec_pd,
                  full((1, E)), full((E, H)), full((1, H)),
                  full((1, E)), full((1, H)), full((1, 128))],
        out_specs=pl.BlockSpec((bb, 128), lambda j: (j, 0)),
        out_shape=jax.ShapeDtypeStruct((NB, 128), jnp.float32),
    )(kk, kk, vv, vv, pepe, pepe, b_edge.reshape(1, E), Wq,
      v_att.reshape(1, H), ws1, ws2, bsv)


def _gather_rows(table, idx):
    nrow, d = table.shape
    m = idx.shape[0]
    nw = 32
    bpw = m // nw
    ch = bpw
    while ch * d * 4 > 131072:
        ch //= 2
    nch = bpw // ch
    mesh = plsc.VectorSubcoreMesh(core_axis_name="c", subcore_axis_name="s")

    @functools.partial(
        pl.kernel, mesh=mesh,
        out_type=jax.ShapeDtypeStruct((m, d), table.dtype),
        scratch_types=[
            pltpu.VMEM((ch,), jnp.int32),
            pltpu.VMEM((ch, d), table.dtype),
            pltpu.VMEM((ch, d), table.dtype),
            pltpu.SemaphoreType.DMA,
            pltpu.SemaphoreType.DMA,
            pltpu.SemaphoreType.DMA,
        ],
    )
    def k(tab_hbm, idx_hbm, out_hbm, idx_v, rows0, rows1, gsem, ws0, ws1):
        wid = lax.axis_index("s") * 2 + lax.axis_index("c")
        base = wid * bpw
        bufs = (rows0, rows1)
        wsems = (ws0, ws1)
        wb = [None, None]
        for c in range(nch):
            b = c % 2
            off = base + c * ch
            if wb[b] is not None:
                wb[b].wait()
            pltpu.sync_copy(idx_hbm.at[pl.ds(off, ch)], idx_v)
            pltpu.async_copy(tab_hbm.at[idx_v], bufs[b], gsem).wait()
            wb[b] = pltpu.async_copy(bufs[b], out_hbm.at[pl.ds(off, ch)],
                                     wsems[b])
        for b in range(2):
            if wb[b] is not None:
                wb[b].wait()

    return k(table, idx)


def _gather_rows2(ta, tb, idx):
    d = ta.shape[1]
    m = idx.shape[0]
    nw = 32
    bpw = m // nw
    ch = bpw
    while ch * d * 4 > 131072:
        ch //= 2
    nch = bpw // ch
    mesh = plsc.VectorSubcoreMesh(core_axis_name="c", subcore_axis_name="s")

    @functools.partial(
        pl.kernel, mesh=mesh,
        out_type=[jax.ShapeDtypeStruct((m, d), ta.dtype),
                  jax.ShapeDtypeStruct((m, d), tb.dtype)],
        scratch_types=[
            pltpu.VMEM((ch,), jnp.int32),
            pltpu.VMEM((ch, d), ta.dtype),
            pltpu.VMEM((ch, d), tb.dtype),
            pltpu.SemaphoreType.DMA,
            pltpu.SemaphoreType.DMA,
            pltpu.SemaphoreType.DMA,
        ],
    )
    def k(ta_hbm, tb_hbm, idx_hbm, oa_hbm, ob_hbm,
          idx_v, rows_a, rows_b, gsem, wsa, wsb):
        wid = lax.axis_index("s") * 2 + lax.axis_index("c")
        base = wid * bpw
        wa = wb = None
        for c in range(nch):
            off = base + c * ch
            pltpu.sync_copy(idx_hbm.at[pl.ds(off, ch)], idx_v)
            if wa is not None:
                wa.wait()
            pltpu.async_copy(ta_hbm.at[idx_v], rows_a, gsem).wait()
            wa = pltpu.async_copy(rows_a, oa_hbm.at[pl.ds(off, ch)], wsa)
            if wb is not None:
                wb.wait()
            pltpu.async_copy(tb_hbm.at[idx_v], rows_b, gsem).wait()
            wb = pltpu.async_copy(rows_b, ob_hbm.at[pl.ds(off, ch)], wsb)
        wa.wait()
        wb.wait()

    return k(ta, tb, idx)


def kernel(mm_f, mm_s, mm_g, dd_t, dd_s, dd_g, m_d, md_node, Wp_m, c_m, Wp_d,
           c_d, Wm, Wd, w_rel, W_agg, b_agg, W_edge, b_edge, Wq, Wk, Wv, v_att,
           W_s, b_s):
    def pad128(v):
        return jnp.zeros((1, 128), jnp.float32).at[0, :v.shape[0]].set(v)

    def _view_combine(views, Wp, c):
        stack = jnp.stack(views)
        h = jnp.tanh(jnp.einsum('vij,jk->vik', stack, Wp))
        s = jnp.mean(jnp.einsum('vik,k->vi', h, c), axis=1)
        return jnp.einsum('v,vij->ij', jax.nn.softmax(s), stack)

    spu_m = _sparsify(_view_combine((mm_f, mm_s, mm_g), Wp_m, c_m), M)
    spu_d = _sparsify(_view_combine((dd_t, dd_s, dd_g), Wp_d, c_d), D)

    sp_m, af_m, pe_m = _featpe(spu_m, spu_m.T, Wm, W_edge[:F], M)
    sp_d, af_d, pe_d = _featpe(spu_d, spu_d.T, Wd, W_edge[F:], D)
    all_feat = jnp.concatenate([af_m, af_d], 0)
    pe = jnp.concatenate([pe_m, pe_d], 0)

    t8m, rel8m = _top8(sp_m, m_d, M, 0)
    t8d, rel8d = _top8(m_d.T, sp_d, D, M)
    t8p = jnp.concatenate([t8m, t8d], 0)
    rel8p = jnp.concatenate([rel8m, rel8d], 0)
    t8f = t8p[:, :S].reshape(NA * S)

    af8 = _gather_rows(all_feat, t8f).reshape(NA, S, F)
    agg = _agg(all_feat, af8, rel8p, pad128(w_rel), W_agg, b_agg.reshape(1, F))

    NU = D + D
    t8u = jnp.concatenate([t8p[:D, :S], t8p[M:, :S]], 0).reshape(NU * S)
    rel8u = jnp.concatenate([rel8p[:D, :S], rel8p[M:, :S]], 0).reshape(NU * S)
    agg8 = _gather_rows(agg, t8u)
    roh = jax.nn.one_hot(rel8u, 8, dtype=jnp.float32)
    wkr8 = jnp.zeros((8, H), jnp.float32).at[:NR].set(Wk[:NR])
    wvr8 = jnp.zeros((8, H), jnp.float32).at[:NR].set(Wv[:NR])
    kr, vr = _krvr(agg8, roh, Wk[NR:], Wv[NR:], wkr8, wvr8)
    kr2 = kr.reshape(NU, S * H)
    vr2 = vr.reshape(NU, S * H)
    peu = jnp.concatenate([pe[:D], pe[M:]], 0)

    idx = jnp.concatenate([md_node[:, 0], md_node[:, 1] + D])
    kk, vv = _gather_rows2(kr2, vr2, idx)
    pepe = _gather_rows(peu, idx)

    out = _final(kk, vv, pepe, b_edge, Wq, v_att,
                 W_s[:E].reshape(1, E), W_s[E:].reshape(1, H), pad128(b_s))
    return out[:, 0]
